# baseline (speedup 1.0000x reference)
import jax
import jax.numpy as jnp
from jax.experimental import pallas as pl
from jax.experimental.pallas import tpu as pltpu

_CH = 16
_D = 8
_K = 4


def _body(emb_ref, in_hbm, out_hbm, *scratch):
    bufs = scratch[:_D]
    isems = scratch[_D : 2 * _D]
    osems = scratch[2 * _D :]
    batch, rows, dim = in_hbm.shape
    n_emb = emb_ref.shape[0]
    n = batch // _CH

    for i in range(_D):
        bufs[i][:, :n_emb, :] = jnp.broadcast_to(
            emb_ref[...][None, :, :], (_CH, n_emb, dim)
        )

    def in_copy(g):
        return pltpu.make_async_copy(
            in_hbm.at[pl.ds(g * _CH, _CH)],
            bufs[g % _D].at[:, pl.ds(n_emb, rows)],
            isems[g % _D],
        )

    def out_copy(g):
        return pltpu.make_async_copy(
            bufs[g % _D],
            out_hbm.at[pl.ds(g * _CH, _CH)],
            osems[g % _D],
        )

    for g in range(_D - _K):
        in_copy(g).start()
    for g in range(n):
        if g >= _K:
            out_copy(g - _K).wait()
        nxt = g + _D - _K
        if nxt < n:
            in_copy(nxt).start()
        in_copy(g).wait()
        out_copy(g).start()
    for g in range(n - _K, n):
        out_copy(g).wait()


@jax.jit
def kernel(inputs, emb_table):
    batch, rows, dim = inputs.shape
    n_emb = emb_table.shape[0]
    out_rows = rows + n_emb
    out_shape = jax.ShapeDtypeStruct((batch, out_rows, dim), inputs.dtype)
    return pl.pallas_call(
        _body,
        out_shape=out_shape,
        in_specs=[
            pl.BlockSpec(memory_space=pltpu.VMEM),
            pl.BlockSpec(memory_space=pltpu.MemorySpace.HBM),
        ],
        out_specs=pl.BlockSpec(memory_space=pltpu.MemorySpace.HBM),
        scratch_shapes=(
            [pltpu.VMEM((_CH, out_rows, dim), jnp.float32) for _ in range(_D)]
            + [pltpu.SemaphoreType.DMA for _ in range(2 * _D)]
        ),
    )(emb_table, inputs)
